# Initial kernel scaffold; baseline (speedup 1.0000x reference)
import jax
import jax.numpy as jnp
from jax.experimental import pallas as pl


def kernel(x, edge_index, W1, b1, W2, b2):
    raise NotImplementedError("write your pallas kernel here")


# speedup vs baseline: 24.9210x; 24.9210x over previous
import functools

import jax
import jax.numpy as jnp
from jax import lax
from jax.experimental import pallas as pl
from jax.experimental.pallas import tpu as pltpu
from jax.experimental.pallas import tpu_sc as plsc

N = 10000
E = 320000
NC = 2
NS = 16
NT = NC * NS
K = 80
TROWS = E // (NT * K)
F = 64
WB = 624
WCH = 48
DEGF = 16

_mesh = plsc.VectorSubcoreMesh(core_axis_name="c", subcore_axis_name="s")


def _zero_fill(buf, rows, width):
  def body(i, carry):
    for j in range(width // 16):
      buf[i, pl.ds(j * 16, 16)] = jnp.zeros((16,), jnp.float32)
    return carry
  lax.fori_loop(0, rows, body, 0)


def _init_acc(acc, zbuf, s):
  for k in range(WB // WCH):
    pltpu.sync_copy(zbuf, acc.at[pl.ds(s * WB + k * WCH, WCH)])

  @pl.when(s == NS - 1)
  def _():
    pltpu.sync_copy(zbuf.at[pl.ds(0, 16)], acc.at[pl.ds(NS * WB, 16)])


def _write_out(acc, out_hbm, c, s):
  for k in range(WB // WCH):
    r = s * WB + k * WCH
    pltpu.sync_copy(acc.at[pl.ds(r, WCH)], out_hbm.at[c, pl.ds(r, WCH)])

  @pl.when(s == NS - 1)
  def _():
    pltpu.sync_copy(acc.at[pl.ds(NS * WB, 16)],
                    out_hbm.at[c, pl.ds(NS * WB, 16)])


def _sc_prop(npass):
  @functools.partial(
      pl.kernel,
      out_type=tuple(
          jax.ShapeDtypeStruct((NC, N, F), jnp.float32) for _ in range(npass)),
      mesh=_mesh,
      compiler_params=pltpu.CompilerParams(use_tc_tiling_on_sc=False),
      scratch_types=[
          pltpu.VMEM_SHARED((N, F), jnp.float32),
          pltpu.VMEM((TROWS, K), jnp.int32),
          pltpu.VMEM((TROWS, K), jnp.int32),
          pltpu.VMEM((2, K, F), jnp.float32),
          pltpu.VMEM((WCH, F), jnp.float32),
          pltpu.SemaphoreType.DMA,
      ],
  )
  def prop(*args):
    gs = args[:npass]
    row_hbm, col_hbm = args[npass], args[npass + 1]
    outs = args[npass + 2:npass + 2 + npass]
    acc, row_v, col_v, gbuf, zbuf, gsem = args[npass + 2 + npass:]

    c = lax.axis_index("c")
    s = lax.axis_index("s")
    t = c * NS + s

    _zero_fill(zbuf, WCH, F)
    pltpu.sync_copy(row_hbm.at[t], row_v)
    pltpu.sync_copy(col_hbm.at[t], col_v)

    for p in range(npass):
      g_hbm, out_hbm = gs[p], outs[p]
      _init_acc(acc, zbuf, s)
      plsc.subcore_barrier()

      pltpu.async_copy(g_hbm.at[row_v.at[0]], gbuf.at[0], gsem)

      def chunk(j, carry):
        slot = lax.rem(j, 2)
        nxt = lax.rem(j + 1, 2)

        @pl.when(j + 1 < TROWS)
        def _():
          pltpu.async_copy(g_hbm.at[row_v.at[j + 1]], gbuf.at[nxt], gsem)

        pltpu.make_async_copy(
            g_hbm.at[row_v.at[j]], gbuf.at[slot], gsem).wait()
        pltpu.sync_copy(gbuf.at[slot], acc.at[col_v.at[j]], add=True)
        return carry

      lax.fori_loop(0, TROWS, chunk, 0)

      plsc.subcore_barrier()
      _write_out(acc, out_hbm, c, s)

  return prop


_prop2 = _sc_prop(2)
_prop1 = _sc_prop(1)


@functools.partial(
    pl.kernel,
    out_type=jax.ShapeDtypeStruct((NC, N, DEGF), jnp.float32),
    mesh=_mesh,
    compiler_params=pltpu.CompilerParams(use_tc_tiling_on_sc=False),
    scratch_types=[
        pltpu.VMEM_SHARED((N, DEGF), jnp.float32),
        pltpu.VMEM((TROWS, K), jnp.int32),
        pltpu.VMEM((K, DEGF), jnp.float32),
        pltpu.VMEM((WCH, DEGF), jnp.float32),
    ],
)
def _sc_degree(col_hbm, out_hbm, acc, col_v, ones_v, zbuf):
  c = lax.axis_index("c")
  s = lax.axis_index("s")
  t = c * NS + s

  _zero_fill(zbuf, WCH, DEGF)

  def fill_ones(i, carry):
    ones_v[i, pl.ds(0, 16)] = jnp.ones((16,), jnp.float32)
    return carry
  lax.fori_loop(0, K, fill_ones, 0)

  _init_acc(acc, zbuf, s)
  pltpu.sync_copy(col_hbm.at[t], col_v)
  plsc.subcore_barrier()

  def chunk(j, carry):
    pltpu.sync_copy(ones_v, acc.at[col_v.at[j]], add=True)
    return carry
  lax.fori_loop(0, TROWS, chunk, 0)

  plsc.subcore_barrier()
  _write_out(acc, out_hbm, c, s)


_BLK = 1000


def _dinv_of(degp_ref):
  deg = degp_ref[0, :, 0] + degp_ref[1, :, 0] + 1.0
  return lax.rsqrt(deg)


def _tc_pre_body(x_ref, w_ref, degp_ref, ga_ref, gb_ref):
  dinv = _dinv_of(degp_ref)
  h = jnp.dot(x_ref[...], w_ref[...], preferred_element_type=jnp.float32)
  g = h * dinv[:, None]
  ga_ref[...] = g[:, :F]
  gb_ref[...] = g[:, F:]


def _tc_mid_body(acca_ref, accb_ref, ga_ref, gb_ref, degp_ref, b_ref, w_ref,
                 out_ref):
  dinv = _dinv_of(degp_ref)
  ha = acca_ref[0] + acca_ref[1] + ga_ref[...]
  hb = accb_ref[0] + accb_ref[1] + gb_ref[...]
  h = jnp.concatenate([ha, hb], axis=1) * dinv[:, None] + b_ref[...][None, :]
  h = jnp.maximum(h, 0.0)
  out_ref[...] = jnp.dot(
      h, w_ref[...], preferred_element_type=jnp.float32) * dinv[:, None]


def _tc_post_body(acc_ref, g_ref, degp_ref, b_ref, out_ref):
  dinv = _dinv_of(degp_ref)
  out_ref[...] = (acc_ref[0] + acc_ref[1] + g_ref[...]) * dinv[:, None] \
      + b_ref[...][None, :]


def _row_blocked(feat):
  return pl.BlockSpec((_BLK, feat), lambda i: (i, 0))


def _acc_blocked(feat):
  return pl.BlockSpec((NC, _BLK, feat), lambda i: (0, i, 0))


_degp_spec = pl.BlockSpec((NC, _BLK, DEGF), lambda i: (0, i, 0))


def _full(shape):
  return pl.BlockSpec(shape, lambda i: tuple(0 for _ in shape))


def kernel(x, edge_index, W1, b1, W2, b2):
  row = edge_index[0].astype(jnp.int32).reshape(NT, TROWS, K)
  col = edge_index[1].astype(jnp.int32).reshape(NT, TROWS, K)

  degp = _sc_degree(col)

  ga, gb = pl.pallas_call(
      _tc_pre_body,
      grid=(N // _BLK,),
      in_specs=[_row_blocked(128), _full((128, 128)), _degp_spec],
      out_specs=[_row_blocked(F), _row_blocked(F)],
      out_shape=[jax.ShapeDtypeStruct((N, F), jnp.float32)] * 2,
  )(x, W1, degp)

  acca, accb = _prop2(ga, gb, row, col)

  g2 = pl.pallas_call(
      _tc_mid_body,
      grid=(N // _BLK,),
      in_specs=[_acc_blocked(F), _acc_blocked(F), _row_blocked(F),
                _row_blocked(F), _degp_spec, _full((128,)), _full((128, 64))],
      out_specs=_row_blocked(64),
      out_shape=jax.ShapeDtypeStruct((N, 64), jnp.float32),
  )(acca, accb, ga, gb, degp, b1, W2)

  (acc2,) = _prop1(g2, row, col)

  out = pl.pallas_call(
      _tc_post_body,
      grid=(N // _BLK,),
      in_specs=[_acc_blocked(64), _row_blocked(64), _degp_spec, _full((64,))],
      out_specs=_row_blocked(64),
      out_shape=jax.ShapeDtypeStruct((N, 64), jnp.float32),
  )(acc2, g2, degp, b2)

  return out
